# Initial kernel scaffold; baseline (speedup 1.0000x reference)
import jax
import jax.numpy as jnp
from jax.experimental import pallas as pl


def kernel(x, edge_index, edge_weight, W1, b1, Wh1, bh1, W2, b2, Wh2, bh2):
    raise NotImplementedError("write your pallas kernel here")


# speedup vs baseline: 4.2835x; 4.2835x over previous
import functools

import jax
import jax.numpy as jnp
from jax import lax
from jax.experimental import pallas as pl
from jax.experimental.pallas import tpu as pltpu
from jax.experimental.pallas import tpu_sc as plsc

N = 10000
E = 320000
F = 128

NC = 2
NS = 16
NW = NC * NS
K = 128
NCHUNK = 79
EPW = NCHUNK * K
EPAD = EPW * NW
ACC_ROWS = 10240
SINK = N + 8
ZROWS = 640


def _sc_edge_agg(h, src3, dst3, zeros):
    mesh = plsc.VectorSubcoreMesh(core_axis_name="c", subcore_axis_name="s",
                                  num_cores=NC, num_subcores=NS)

    @functools.partial(
        pl.kernel,
        mesh=mesh,
        out_type=jax.ShapeDtypeStruct((NC, ACC_ROWS, F), jnp.float32),
        scratch_types=[
            pltpu.VMEM((NCHUNK, K), jnp.int32),
            pltpu.VMEM((NCHUNK, K), jnp.int32),
            pltpu.VMEM((K, F), jnp.float32),
            pltpu.VMEM_SHARED((ACC_ROWS, F), jnp.float32),
            pltpu.SemaphoreType.DMA,
        ],
    )
    def agg(h_hbm, src_hbm, dst_hbm, zero_hbm, out_hbm, src_v, dst_v, rows_v,
            acc_sh, sem):
        cid = lax.axis_index("c")
        sid = lax.axis_index("s")
        wid = sid * NC + cid

        pltpu.sync_copy(zero_hbm, acc_sh.at[pl.ds(sid * ZROWS, ZROWS)])
        pltpu.sync_copy(src_hbm.at[wid], src_v)
        pltpu.sync_copy(dst_hbm.at[wid], dst_v)
        plsc.subcore_barrier()

        def body(j, _):
            pltpu.async_copy(h_hbm.at[src_v.at[j]], rows_v, sem).wait()
            pltpu.sync_copy(rows_v, acc_sh.at[dst_v.at[j]], add=True)
            return 0

        lax.fori_loop(0, NCHUNK, body, 0)
        plsc.subcore_barrier()

        pltpu.sync_copy(acc_sh.at[pl.ds(sid * ZROWS, ZROWS)],
                        out_hbm.at[cid, pl.ds(sid * ZROWS, ZROWS)])

    return agg(h, src3, dst3, zeros)[:, :N, :]


def _mm_body(x_ref, w_ref, o_ref):
    o_ref[...] = jnp.dot(x_ref[...], w_ref[...],
                         preferred_element_type=jnp.float32)


def _matmul(x, w):
    bm = 1000
    return pl.pallas_call(
        _mm_body,
        grid=(N // bm,),
        in_specs=[
            pl.BlockSpec((bm, F), lambda i: (i, 0)),
            pl.BlockSpec((F, F), lambda i: (0, 0)),
        ],
        out_specs=pl.BlockSpec((bm, F), lambda i: (i, 0)),
        out_shape=jax.ShapeDtypeStruct((N, F), jnp.float32),
    )(x, w)


def _mid_body(p_ref, h_ref, b1_ref, wh1_ref, bh1_ref, w2_ref, o_ref):
    r = jax.nn.relu(p_ref[0] + p_ref[1] + h_ref[...] + b1_ref[...])
    t = jnp.dot(r, wh1_ref[...], preferred_element_type=jnp.float32)
    t = t + bh1_ref[...]
    o_ref[...] = jnp.dot(t, w2_ref[...], preferred_element_type=jnp.float32)


def _mid(p, h1, b1, wh1, bh1, w2):
    bm = 1000
    return pl.pallas_call(
        _mid_body,
        grid=(N // bm,),
        in_specs=[
            pl.BlockSpec((NC, bm, F), lambda i: (0, i, 0)),
            pl.BlockSpec((bm, F), lambda i: (i, 0)),
            pl.BlockSpec((1, F), lambda i: (0, 0)),
            pl.BlockSpec((F, F), lambda i: (0, 0)),
            pl.BlockSpec((1, F), lambda i: (0, 0)),
            pl.BlockSpec((F, F), lambda i: (0, 0)),
        ],
        out_specs=pl.BlockSpec((bm, F), lambda i: (i, 0)),
        out_shape=jax.ShapeDtypeStruct((N, F), jnp.float32),
    )(p, h1, b1.reshape(1, F), wh1, bh1.reshape(1, F), w2)


def _final_body(q_ref, h2_ref, b2_ref, wh2_ref, bh2_ref, o_ref):
    a = q_ref[0] + q_ref[1] + h2_ref[...] + b2_ref[...]
    o = jnp.dot(a, wh2_ref[...], preferred_element_type=jnp.float32)
    o = o + bh2_ref[...]
    m = jnp.max(o, axis=1, keepdims=True)
    e = o - m
    lse = jnp.log(jnp.sum(jnp.exp(e), axis=1, keepdims=True))
    o_ref[...] = e - lse


def _final(q, h2, b2, wh2, bh2):
    bm = 1000
    return pl.pallas_call(
        _final_body,
        grid=(N // bm,),
        in_specs=[
            pl.BlockSpec((NC, bm, F), lambda i: (0, i, 0)),
            pl.BlockSpec((bm, F), lambda i: (i, 0)),
            pl.BlockSpec((1, F), lambda i: (0, 0)),
            pl.BlockSpec((F, F), lambda i: (0, 0)),
            pl.BlockSpec((1, F), lambda i: (0, 0)),
        ],
        out_specs=pl.BlockSpec((bm, F), lambda i: (i, 0)),
        out_shape=jax.ShapeDtypeStruct((N, F), jnp.float32),
    )(q, h2, b2.reshape(1, F), wh2, bh2.reshape(1, F))


def kernel(x, edge_index, edge_weight, W1, b1, Wh1, bh1, W2, b2, Wh2, bh2):
    del edge_weight
    src = edge_index[0]
    dst = edge_index[1]
    pad = EPAD - E
    src3 = jnp.concatenate([src, jnp.zeros((pad,), jnp.int32)]).reshape(
        NW, NCHUNK, K)
    dst3 = jnp.concatenate([dst, jnp.full((pad,), SINK, jnp.int32)]).reshape(
        NW, NCHUNK, K)
    zeros = jnp.zeros((ZROWS, F), jnp.float32)

    h1 = _matmul(x, W1)
    p = _sc_edge_agg(h1, src3, dst3, zeros)
    h2 = _mid(p, h1, b1, Wh1, bh1, W2)
    q = _sc_edge_agg(h2, src3, dst3, zeros)
    return _final(q, h2, b2, Wh2, bh2)
